# baseline (speedup 1.0000x reference)
import functools

import jax
import jax.numpy as jnp
from jax import lax
from jax.experimental import pallas as pl
from jax.experimental.pallas import tpu as pltpu
from jax.experimental.pallas import tpu_sc as plsc

B = 16384
D = 32
NC = 2
NS = 16
NW = NC * NS
BPW = B // NW

_sc_mesh = plsc.VectorSubcoreMesh(core_axis_name="c", subcore_axis_name="s")


@functools.partial(
    pl.kernel,
    mesh=_sc_mesh,
    compiler_params=pltpu.CompilerParams(use_tc_tiling_on_sc=False),
    out_type=[
        jax.ShapeDtypeStruct((B, D), jnp.float32),
        jax.ShapeDtypeStruct((B, D), jnp.float32),
    ],
    scratch_types=[
        pltpu.VMEM((BPW,), jnp.int32),
        pltpu.VMEM((BPW,), jnp.int32),
        pltpu.VMEM((BPW, D), jnp.float32),
        pltpu.VMEM((BPW, D), jnp.float32),
        pltpu.SemaphoreType.DMA,
    ],
)
def _sc_gather(user_hbm, item_hbm, ut_hbm, it_hbm, ue_hbm, ie_hbm,
               uidx_v, iidx_v, urows_v, irows_v, sem):
    wid = lax.axis_index("s") * NC + lax.axis_index("c")
    base = wid * BPW
    pltpu.sync_copy(user_hbm.at[pl.ds(base, BPW)], uidx_v)
    pltpu.sync_copy(item_hbm.at[pl.ds(base, BPW)], iidx_v)
    cu = pltpu.async_copy(ut_hbm.at[uidx_v], urows_v, sem)
    ci = pltpu.async_copy(it_hbm.at[iidx_v], irows_v, sem)
    cu.wait()
    ci.wait()
    pltpu.sync_copy(urows_v, ue_hbm.at[pl.ds(base, BPW)])
    pltpu.sync_copy(irows_v, ie_hbm.at[pl.ds(base, BPW)])


TB = 2048


def _tc_dense_body(ue_ref, ie_ref, wut_ref, wit_ref, w1t_ref, b1_ref,
                   w2t_ref, b2_ref, w3t_ref, bias_ref, out_ref):
    ue = ue_ref[...]
    ie = ie_ref[...]
    inter = ue * ie
    fm = (jnp.dot(ue, wut_ref[...], preferred_element_type=jnp.float32)
          + jnp.dot(ie, wit_ref[...], preferred_element_type=jnp.float32))
    h = jnp.maximum(
        jnp.dot(inter, w1t_ref[...], preferred_element_type=jnp.float32)
        + b1_ref[...], 0.0)
    h = jnp.maximum(
        jnp.dot(h, w2t_ref[...], preferred_element_type=jnp.float32)
        + b2_ref[...], 0.0)
    deep = jnp.dot(h, w3t_ref[...], preferred_element_type=jnp.float32)
    logit = fm[:, 0] + deep[:, 0] + bias_ref[0]
    out_ref[...] = 1.0 / (1.0 + jnp.exp(-logit))


def _tc_dense(ue, ie, wut, wit, w1t, b1, w2t, b2, w3t, bias):
    grid = (B // TB,)
    return pl.pallas_call(
        _tc_dense_body,
        grid=grid,
        in_specs=[
            pl.BlockSpec((TB, D), lambda i: (i, 0)),
            pl.BlockSpec((TB, D), lambda i: (i, 0)),
            pl.BlockSpec(wut.shape, lambda i: (0, 0)),
            pl.BlockSpec(wit.shape, lambda i: (0, 0)),
            pl.BlockSpec(w1t.shape, lambda i: (0, 0)),
            pl.BlockSpec(b1.shape, lambda i: (0,)),
            pl.BlockSpec(w2t.shape, lambda i: (0, 0)),
            pl.BlockSpec(b2.shape, lambda i: (0,)),
            pl.BlockSpec(w3t.shape, lambda i: (0, 0)),
            pl.BlockSpec(bias.shape, lambda i: (0,)),
        ],
        out_specs=pl.BlockSpec((TB,), lambda i: (i,)),
        out_shape=jax.ShapeDtypeStruct((B,), jnp.float32),
    )(ue, ie, wut, wit, w1t, b1, w2t, b2, w3t, bias)


def kernel(user, item, user_table, item_table, fm_W, fm_b, W1, b1, W2, b2, W3, b3):
    user = user.astype(jnp.int32)
    item = item.astype(jnp.int32)
    ue, ie = _sc_gather(user, item, user_table, item_table)
    wut = fm_W[:, :D].T
    wit = fm_W[:, D:].T
    bias = (fm_b + b3).reshape((1,))
    return _tc_dense(ue, ie, wut, wit, W1.T, b1, W2.T, b2, W3.T, bias)
